# Initial kernel scaffold; baseline (speedup 1.0000x reference)
import jax
import jax.numpy as jnp
from jax.experimental import pallas as pl


def kernel(x, router_W, router_b, exp_Wgate, exp_Wup, exp_Wdown, share_up_W, share_up_b, share_down_W, share_down_b, swiglu_W, swiglu_b):
    raise NotImplementedError("write your pallas kernel here")


# speedup vs baseline: 1.3808x; 1.3808x over previous
import jax
import jax.numpy as jnp
from jax.experimental import pallas as pl
from jax.experimental.pallas import tpu as pltpu

HID = 2048
E = 8
NR = 6
EXPD = 1024
N = 4096
BM = 512
BM3 = 1024


def _router_kernel(h_ref, wr_ref, br_ref, w_ref):
    h = h_ref[...]
    gate = jax.lax.dot_general(
        h, wr_ref[...], (((1,), (1,)), ((), ())),
        preferred_element_type=jnp.float32)
    gate = gate + br_ref[...]
    cols = jax.lax.broadcasted_iota(jnp.int32, gate.shape, 1)
    m = jnp.max(gate, axis=1, keepdims=True)
    ex_cols = []
    for i in range(E):
        gi = gate[:, i:i + 1]
        greater = (gate > gi).astype(jnp.int32)
        eq_lower = ((gate == gi) & (cols < i)).astype(jnp.int32)
        cnt = jnp.sum(greater + eq_lower, axis=1, keepdims=True)
        sel = (cnt < NR).astype(jnp.float32)
        ex_cols.append(sel * jnp.exp(gi - m))
    ex = jnp.concatenate(ex_cols, axis=1)
    w_ref[...] = ex / jnp.sum(ex, axis=1, keepdims=True)


def _gateup_kernel(w_ref, h_ref, wg_ref, wu_ref, a_ref):
    e = pl.program_id(0)
    h = h_ref[...]
    g = jax.lax.dot_general(
        h, wg_ref[0], (((1,), (1,)), ((), ())),
        preferred_element_type=jnp.float32)
    u = jax.lax.dot_general(
        h, wu_ref[0], (((1,), (1,)), ((), ())),
        preferred_element_type=jnp.float32)
    wcols = w_ref[...]
    onehot = (jax.lax.broadcasted_iota(jnp.int32, wcols.shape, 1) == e)
    we = jnp.sum(jnp.where(onehot, wcols, 0.0), axis=1, keepdims=True)
    a_ref[...] = (jax.nn.silu(g) * u) * we


def _down_kernel(a_ref, wd_ref, out_ref):
    e = pl.program_id(1)
    part = jax.lax.dot_general(
        a_ref[...], wd_ref[0], (((1,), (1,)), ((), ())),
        preferred_element_type=jnp.float32)

    @pl.when(e == 0)
    def _():
        out_ref[...] = part

    @pl.when(e != 0)
    def _():
        out_ref[...] += part


def _sharedup_kernel(h_ref, wup_ref, bup_ref, wsw_ref, bsw_ref, sact_ref):
    h = h_ref[...]
    s = jax.lax.dot_general(
        h, wup_ref[...], (((1,), (1,)), ((), ())),
        preferred_element_type=jnp.float32) + bup_ref[...]
    s0 = s[:, :EXPD]
    s1 = s[:, EXPD:]
    sw0 = jax.lax.dot_general(
        s0, wsw_ref[...], (((1,), (1,)), ((), ())),
        preferred_element_type=jnp.float32) + bsw_ref[...]
    sw1 = jax.lax.dot_general(
        s1, wsw_ref[...], (((1,), (1,)), ((), ())),
        preferred_element_type=jnp.float32) + bsw_ref[...]
    a0 = jax.nn.silu(sw0[:, :EXPD]) * sw0[:, EXPD:]
    a1 = jax.nn.silu(sw1[:, :EXPD]) * sw1[:, EXPD:]
    sact_ref[...] = jnp.concatenate([a0, a1], axis=1)


def _combine_kernel(x_ref, routed_ref, sact_ref, wdn_ref, bdn_ref, out_ref):
    out = jax.lax.dot_general(
        sact_ref[...], wdn_ref[...], (((1,), (1,)), ((), ())),
        preferred_element_type=jnp.float32) + bdn_ref[...]
    out_ref[...] = x_ref[...] + routed_ref[...] + out


def _chain(x, h, rW, rb, exp_Wgate, exp_Wup, exp_Wdown,
           share_up_W, share_up_b, share_down_W, share_down_b,
           swiglu_W, swiglu_b):
    nt = N // BM
    arb = pltpu.CompilerParams(dimension_semantics=("arbitrary",))
    arb2 = pltpu.CompilerParams(dimension_semantics=("arbitrary", "arbitrary"))

    w = pl.pallas_call(
        _router_kernel,
        grid=(nt,),
        in_specs=[
            pl.BlockSpec((BM, HID), lambda t: (t, 0)),
            pl.BlockSpec((E, HID), lambda t: (0, 0)),
            pl.BlockSpec((1, E), lambda t: (0, 0)),
        ],
        out_specs=pl.BlockSpec((BM, E), lambda t: (t, 0)),
        out_shape=jax.ShapeDtypeStruct((N, E), jnp.float32),
        compiler_params=arb,
    )(h, rW, rb.reshape(1, E))

    a = pl.pallas_call(
        _gateup_kernel,
        grid=(NR, nt),
        in_specs=[
            pl.BlockSpec((BM, E), lambda e, t: (t, 0)),
            pl.BlockSpec((BM, HID), lambda e, t: (t, 0)),
            pl.BlockSpec((1, EXPD, HID), lambda e, t: (e, 0, 0)),
            pl.BlockSpec((1, EXPD, HID), lambda e, t: (e, 0, 0)),
        ],
        out_specs=pl.BlockSpec((BM, EXPD), lambda e, t: (t, e)),
        out_shape=jax.ShapeDtypeStruct((N, NR * EXPD), jnp.float32),
        compiler_params=arb2,
    )(w, h, exp_Wgate, exp_Wup)

    routed = pl.pallas_call(
        _down_kernel,
        grid=(N // BM3, NR),
        in_specs=[
            pl.BlockSpec((BM3, EXPD), lambda t, e: (t, e)),
            pl.BlockSpec((1, HID, EXPD), lambda t, e: (e, 0, 0)),
        ],
        out_specs=pl.BlockSpec((BM3, HID), lambda t, e: (t, 0)),
        out_shape=jax.ShapeDtypeStruct((N, HID), jnp.float32),
        compiler_params=arb2,
    )(a, exp_Wdown)

    sact = pl.pallas_call(
        _sharedup_kernel,
        grid=(nt,),
        in_specs=[
            pl.BlockSpec((BM, HID), lambda t: (t, 0)),
            pl.BlockSpec((2 * EXPD, HID), lambda t: (0, 0)),
            pl.BlockSpec((1, 2 * EXPD), lambda t: (0, 0)),
            pl.BlockSpec((2 * EXPD, EXPD), lambda t: (0, 0)),
            pl.BlockSpec((1, 2 * EXPD), lambda t: (0, 0)),
        ],
        out_specs=pl.BlockSpec((BM, 2 * EXPD), lambda t: (t, 0)),
        out_shape=jax.ShapeDtypeStruct((N, 2 * EXPD), jnp.float32),
        compiler_params=arb,
    )(h, share_up_W, share_up_b.reshape(1, -1), swiglu_W,
      swiglu_b.reshape(1, -1))

    h_next = pl.pallas_call(
        _combine_kernel,
        grid=(nt,),
        in_specs=[
            pl.BlockSpec((BM, HID), lambda t: (t, 0)),
            pl.BlockSpec((BM, HID), lambda t: (t, 0)),
            pl.BlockSpec((BM, 2 * EXPD), lambda t: (t, 0)),
            pl.BlockSpec((HID, 2 * EXPD), lambda t: (0, 0)),
            pl.BlockSpec((1, HID), lambda t: (0, 0)),
        ],
        out_specs=pl.BlockSpec((BM, HID), lambda t: (t, 0)),
        out_shape=jax.ShapeDtypeStruct((N, HID), jnp.float32),
        compiler_params=arb,
    )(x, routed, sact, share_down_W, share_down_b.reshape(1, -1))
    return h_next


def kernel(x, router_W, router_b, exp_Wgate, exp_Wup, exp_Wdown,
           share_up_W, share_up_b, share_down_W, share_down_b,
           swiglu_W, swiglu_b):
    h = x
    for j in range(router_W.shape[0]):
        h = _chain(x, h, router_W[j], router_b[j], exp_Wgate, exp_Wup,
                   exp_Wdown, share_up_W, share_up_b, share_down_W,
                   share_down_b, swiglu_W, swiglu_b)
    return h
